# baseline (speedup 1.0000x reference)
import jax
import jax.numpy as jnp
from jax import lax
from jax.experimental import pallas as pl
from jax.experimental.pallas import tpu as pltpu

_BLK = 1024
_TILE = 128


def _copy_scatter_t(slots, tok_kt, tok_vt, kt, vt):
    n_heads, head_dim, num_slots = kt.shape
    n_tok = tok_kt.shape[2]
    grid = (num_slots // _BLK,)

    def body(slots_ref, kin, vin, tk, tv, kout, vout):
        i = pl.program_id(0)
        base = i * _BLK
        kout[...] = kin[...]
        vout[...] = vin[...]
        for t in range(n_tok):
            s = slots_ref[t]
            sl = s - base
            inb = (s >= base) & (s < base + _BLK)
            for j in range(_BLK // _TILE):
                @pl.when(inb & (sl // _TILE == j))
                def _():
                    lane = sl - j * _TILE
                    mk = lax.broadcasted_iota(
                        jnp.int32, (n_heads, head_dim, _TILE), 2) == lane
                    cols = pl.ds(j * _TILE, _TILE)
                    kout[:, :, cols] = jnp.where(
                        mk, tk[:, :, t:t + 1], kout[:, :, cols])
                    vout[:, :, cols] = jnp.where(
                        mk, tv[:, :, t:t + 1], vout[:, :, cols])

    blk = pl.BlockSpec((n_heads, head_dim, _BLK), lambda i, s: (0, 0, i))
    tokblk = pl.BlockSpec((n_heads, head_dim, n_tok), lambda i, s: (0, 0, 0))
    return pl.pallas_call(
        body,
        grid_spec=pltpu.PrefetchScalarGridSpec(
            num_scalar_prefetch=1,
            grid=grid,
            in_specs=[blk, blk, tokblk, tokblk],
            out_specs=[blk, blk],
        ),
        out_shape=(
            jax.ShapeDtypeStruct(kt.shape, kt.dtype),
            jax.ShapeDtypeStruct(vt.shape, vt.dtype),
        ),
        compiler_params=pltpu.CompilerParams(
            dimension_semantics=("arbitrary",),
        ),
    )(slots, kt, vt, tok_kt, tok_vt)


def kernel(pos_ids, k_val, v_val, slot_mapping, batch_idx, k_cache, v_cache):
    B, H, S, D = k_val.shape
    tok_k = jnp.transpose(k_val, (0, 2, 1, 3)).reshape(B * S, H, D)
    tok_v = jnp.transpose(v_val, (0, 2, 1, 3)).reshape(B * S, H, D)
    kt = jnp.transpose(k_cache, (1, 2, 0))
    vt = jnp.transpose(v_cache, (1, 2, 0))
    tkt = jnp.transpose(tok_k, (1, 2, 0))
    tvt = jnp.transpose(tok_v, (1, 2, 0))
    ko_t, vo_t = _copy_scatter_t(slot_mapping, tkt, tvt, kt, vt)
    return jnp.transpose(ko_t, (2, 0, 1)), jnp.transpose(vo_t, (2, 0, 1))


# speedup vs baseline: 2.9830x; 2.9830x over previous
import jax
import jax.numpy as jnp
from jax.experimental import pallas as pl
from jax.experimental.pallas import tpu as pltpu

_BLK = 2048


def _paged_update(slots, tok_k, tok_v, kc, vc):
    num_slots, row = kc.shape
    n_tok = tok_k.shape[0]
    grid = (num_slots // _BLK,)

    def body(slots_ref, kin, vin, tk, tv, kout, vout):
        i = pl.program_id(0)
        base = i * _BLK
        kout[...] = kin[...]
        vout[...] = vin[...]
        for t in range(n_tok):
            s = slots_ref[t]

            @pl.when((s >= base) & (s < base + _BLK))
            def _():
                r = pl.ds(s - base, 1)
                kout[r, :] = tk[t : t + 1, :]
                vout[r, :] = tv[t : t + 1, :]

    blk = pl.BlockSpec((_BLK, row), lambda i, s: (i, 0))
    tokblk = pl.BlockSpec((n_tok, row), lambda i, s: (0, 0))
    return pl.pallas_call(
        body,
        grid_spec=pltpu.PrefetchScalarGridSpec(
            num_scalar_prefetch=1,
            grid=grid,
            in_specs=[blk, blk, tokblk, tokblk],
            out_specs=[blk, blk],
        ),
        out_shape=(
            jax.ShapeDtypeStruct(kc.shape, kc.dtype),
            jax.ShapeDtypeStruct(vc.shape, vc.dtype),
        ),
        compiler_params=pltpu.CompilerParams(
            dimension_semantics=("arbitrary",),
        ),
    )(slots, kc, vc, tok_k, tok_v)


def kernel(pos_ids, k_val, v_val, slot_mapping, batch_idx, k_cache, v_cache):
    B, H, S, D = k_val.shape
    tok_k = jnp.transpose(k_val, (0, 2, 1, 3)).reshape(B * S, H * D)
    tok_v = jnp.transpose(v_val, (0, 2, 1, 3)).reshape(B * S, H * D)
    kc = k_cache.reshape(k_cache.shape[0], H * D)
    vc = v_cache.reshape(v_cache.shape[0], H * D)
    ko, vo = _paged_update(slot_mapping, tok_k, tok_v, kc, vc)
    return ko.reshape(k_cache.shape), vo.reshape(v_cache.shape)
